# baseline (speedup 1.0000x reference)
import functools

import jax
import jax.numpy as jnp
from jax import lax
from jax.experimental import pallas as pl
from jax.experimental.pallas import tpu as pltpu
from jax.experimental.pallas import tpu_sc as plsc

T = 2048
IN_F = 2048
OUT_F = 8192
BLK = 512
NB = OUT_F // BLK
TOPK = NB // 4

RT = 256
NT = (T * TOPK + NB * RT) // RT
P_MAX = NT * RT
NTP = 64

NW = 32
TPW = T // NW
CCH = 32
TPC = 4

_HI = jax.lax.Precision.HIGHEST


def _meta_body(prob_ref, pos_tok_ref, pos_t_ref, te_ref, xi_ref, tv_ref):
    prob = prob_ref[...]
    ids_e = lax.broadcasted_iota(jnp.int32, (T, NB), 1)

    cols = []
    for e in range(NB):
        pn = prob[:, e:e + 1]
        beats = (prob > pn) | ((prob == pn) & (ids_e < e))
        cnt = jnp.sum(beats.astype(jnp.float32), axis=1, keepdims=True)
        cols.append((cnt < TOPK).astype(jnp.float32))
    maskf = jnp.concatenate(cols, axis=1)
    maskb = maskf > 0.5

    r_i = lax.broadcasted_iota(jnp.int32, (T, T), 0)
    c_i = lax.broadcasted_iota(jnp.int32, (T, T), 1)
    tril = (c_i < r_i).astype(jnp.float32)
    rank = lax.dot_general(tril, maskf, (((1,), (0,)), ((), ())),
                           precision=_HI)

    ones_row = jnp.ones((1, T), jnp.float32)
    counts = lax.dot_general(ones_row, maskf, (((1,), (0,)), ((), ())),
                             precision=_HI)
    pc = jnp.floor((counts + (RT - 1)) / RT) * RT

    re = lax.broadcasted_iota(jnp.int32, (NB, NB), 0)
    ce = lax.broadcasted_iota(jnp.int32, (NB, NB), 1)
    l16s = (re < ce).astype(jnp.float32)
    starts = lax.dot_general(pc, l16s, (((1,), (0,)), ((), ())),
                             precision=_HI)
    ends = starts + pc

    p_te = starts + rank

    l16i = (re <= ce).astype(jnp.float32)
    ordm = lax.dot_general(maskf, l16i, (((1,), (0,)), ((), ())),
                           precision=_HI)

    pcols = []
    for j in range(TOPK):
        selj = maskb & (ordm == (j + 1))
        pcols.append(jnp.sum(jnp.where(selj, p_te, 0.0), axis=1, keepdims=True))
    pos_tok = jnp.concatenate(pcols, axis=1)
    pos_tok_ref[...] = pos_tok.astype(jnp.int32)

    ident = (r_i == c_i).astype(jnp.float32)
    pos_t = lax.dot_general(pos_tok, ident, (((0,), (0,)), ((), ())),
                            precision=_HI)
    pos_t_ref[...] = pos_t.astype(jnp.int32)

    u = jnp.sum(pc, axis=1, keepdims=True) / RT
    it = lax.broadcasted_iota(jnp.int32, (NTP, NB), 0).astype(jnp.float32)
    texp_raw = jnp.sum((it * RT >= ends).astype(jnp.float32),
                       axis=1, keepdims=True)
    texp_last = jnp.sum(((u - 1.0) * RT >= ends).astype(jnp.float32),
                        axis=1, keepdims=True)
    itcol = lax.broadcasted_iota(jnp.int32, (NTP, 1), 0).astype(jnp.float32)
    valid = itcol < u
    te_ref[...] = jnp.where(valid, texp_raw, texp_last).astype(jnp.int32)
    xi_ref[...] = jnp.minimum(itcol, u - 1.0).astype(jnp.int32)
    tv_ref[...] = valid.astype(jnp.int32)


def _run_meta(prob):
    return pl.pallas_call(
        _meta_body,
        out_shape=[
            jax.ShapeDtypeStruct((T, TOPK), jnp.int32),
            jax.ShapeDtypeStruct((TOPK, T), jnp.int32),
            jax.ShapeDtypeStruct((NTP, 1), jnp.int32),
            jax.ShapeDtypeStruct((NTP, 1), jnp.int32),
            jax.ShapeDtypeStruct((NTP, 1), jnp.int32),
        ],
    )(prob)


def _scatter_body(x_hbm, pos_t_hbm, xs_hbm, xb0, xb1, ib0, ib1, lsem, ssem):
    wid = lax.axis_index("s") * 2 + lax.axis_index("c")
    base = wid * TPW
    pltpu.async_copy(x_hbm.at[pl.ds(base, CCH)], xb0, lsem)
    pltpu.async_copy(x_hbm.at[pl.ds(base + CCH, CCH)], xb1, lsem)
    for j in range(TOPK):
        pltpu.sync_copy(pos_t_hbm.at[j, pl.ds(base, CCH)], ib0.at[j])
        pltpu.sync_copy(pos_t_hbm.at[j, pl.ds(base + CCH, CCH)], ib1.at[j])
    pltpu.make_async_copy(x_hbm.at[pl.ds(base, CCH)], xb0, lsem).wait()
    pltpu.make_async_copy(x_hbm.at[pl.ds(base + CCH, CCH)], xb1, lsem).wait()
    for j in range(TOPK):
        pltpu.async_copy(xb0, xs_hbm.at[ib0.at[j]], ssem)
        pltpu.async_copy(xb1, xs_hbm.at[ib1.at[j]], ssem)
    for j in range(TOPK):
        pltpu.make_async_copy(xb0, xs_hbm.at[ib0.at[j]], ssem).wait()
        pltpu.make_async_copy(xb1, xs_hbm.at[ib1.at[j]], ssem).wait()


def _run_scatter(xp, pos_t):
    mesh = plsc.VectorSubcoreMesh(core_axis_name="c", subcore_axis_name="s")
    f = functools.partial(
        pl.kernel,
        out_type=jax.ShapeDtypeStruct((P_MAX, IN_F // 2), jnp.float32),
        mesh=mesh,
        scratch_types=[
            pltpu.VMEM((CCH, IN_F // 2), jnp.float32),
            pltpu.VMEM((CCH, IN_F // 2), jnp.float32),
            pltpu.VMEM((TOPK, CCH), jnp.int32),
            pltpu.VMEM((TOPK, CCH), jnp.int32),
            pltpu.SemaphoreType.DMA,
            pltpu.SemaphoreType.DMA,
        ],
    )(_scatter_body)
    return f(xp, pos_t)


def _ffn_body(te_ref, xi_ref, tv_ref, xs_ref, w1_ref, b1_ref, w2_ref, ys_ref):
    i = pl.program_id(0)

    @pl.when(tv_ref[i] == 1)
    def _():
        xt = xs_ref[...].astype(jnp.float32)
        h = lax.dot_general(xt, w1_ref[...], (((1,), (1,)), ((), ())),
                            preferred_element_type=jnp.float32)
        h = h + b1_ref[0]
        g = jax.nn.gelu(h)
        ys_ref[...] = lax.dot_general(g, w2_ref[...], (((1,), (1,)), ((), ())),
                                      preferred_element_type=jnp.float32)


def _run_ffn(te, xi, tv, xs, W1, b1r, W2):
    grid_spec = pltpu.PrefetchScalarGridSpec(
        num_scalar_prefetch=3,
        grid=(NT,),
        in_specs=[
            pl.BlockSpec((RT, IN_F), lambda i, te, xi, tv: (xi[i], 0)),
            pl.BlockSpec((BLK, IN_F), lambda i, te, xi, tv: (te[i], 0)),
            pl.BlockSpec((1, 1, BLK), lambda i, te, xi, tv: (te[i], 0, 0)),
            pl.BlockSpec((IN_F, BLK), lambda i, te, xi, tv: (0, te[i])),
        ],
        out_specs=pl.BlockSpec((RT, IN_F), lambda i, te, xi, tv: (xi[i], 0)),
    )
    return pl.pallas_call(
        _ffn_body,
        grid_spec=grid_spec,
        out_shape=jax.ShapeDtypeStruct((P_MAX, IN_F), jnp.float32),
        compiler_params=pltpu.CompilerParams(
            dimension_semantics=("arbitrary",),
        ),
    )(te, xi, tv, xs, W1, b1r, W2)


def _accum(rows, acc, b2v):
    def col(ci, carry):
        off = ci * 16
        for i in range(TPC):
            v = b2v[pl.ds(off, 16)]
            for j in range(TOPK):
                v = v + rows[TOPK * i + j, pl.ds(off, 16)]
            acc[i, pl.ds(off, 16)] = v
        return carry

    lax.fori_loop(0, IN_F // 16, col, 0, unroll=4)


def _combine_body(ys_hbm, pos_flat_hbm, b2_hbm, y_hbm, rows0, rows1, acc,
                  idx0, idx1, b2v, sem0, sem1):
    wid = lax.axis_index("s") * 2 + lax.axis_index("c")
    base = wid * TPW
    nch = TPW // TPC
    pltpu.sync_copy(b2_hbm, b2v)
    pltpu.sync_copy(pos_flat_hbm.at[pl.ds(base * TOPK, TPC * TOPK)], idx0)
    pltpu.async_copy(ys_hbm.at[idx0], rows0, sem0)

    def pair(m, carry):
        ta = base + (2 * m) * TPC
        tb = base + (2 * m + 1) * TPC
        pltpu.sync_copy(pos_flat_hbm.at[pl.ds(tb * TOPK, TPC * TOPK)], idx1)
        pltpu.async_copy(ys_hbm.at[idx1], rows1, sem1)
        pltpu.make_async_copy(ys_hbm.at[idx0], rows0, sem0).wait()
        _accum(rows0, acc, b2v)
        pltpu.sync_copy(acc, y_hbm.at[pl.ds(ta, TPC)])
        @pl.when(m + 1 < nch // 2)
        def _():
            tn = base + (2 * m + 2) * TPC
            pltpu.sync_copy(pos_flat_hbm.at[pl.ds(tn * TOPK, TPC * TOPK)],
                            idx0)
            pltpu.async_copy(ys_hbm.at[idx0], rows0, sem0)
        pltpu.make_async_copy(ys_hbm.at[idx1], rows1, sem1).wait()
        _accum(rows1, acc, b2v)
        pltpu.sync_copy(acc, y_hbm.at[pl.ds(tb, TPC)])
        return carry

    lax.fori_loop(0, nch // 2, pair, 0)


def _run_combine(ys, pos_flat, b2):
    mesh = plsc.VectorSubcoreMesh(core_axis_name="c", subcore_axis_name="s")
    f = functools.partial(
        pl.kernel,
        out_type=jax.ShapeDtypeStruct((T, IN_F), jnp.float32),
        mesh=mesh,
        scratch_types=[
            pltpu.VMEM((TPC * TOPK, IN_F), jnp.float32),
            pltpu.VMEM((TPC * TOPK, IN_F), jnp.float32),
            pltpu.VMEM((TPC, IN_F), jnp.float32),
            pltpu.VMEM((TPC * TOPK,), jnp.int32),
            pltpu.VMEM((TPC * TOPK,), jnp.int32),
            pltpu.VMEM((IN_F,), jnp.float32),
            pltpu.SemaphoreType.DMA,
            pltpu.SemaphoreType.DMA,
        ],
    )(_combine_body)
    return f(ys, pos_flat, b2)


def kernel(x, Wr, br, W1, b1, W2, b2):
    logits = x @ Wr.T + br[None, :]
    prob = jax.nn.softmax(logits, axis=-1)

    pos_tok, pos_t, te, xi, tv = _run_meta(prob)
    te = te.reshape(NTP)
    xi = xi.reshape(NTP)
    tv = tv.reshape(NTP)

    xb = x.astype(jnp.bfloat16)
    xp = lax.bitcast_convert_type(xb.reshape(T, IN_F // 2, 2), jnp.float32)
    xsp = _run_scatter(xp, pos_t)
    xs = lax.bitcast_convert_type(xsp, jnp.bfloat16).reshape(P_MAX, IN_F)

    b1r = b1.reshape(NB, 1, BLK)
    ys = _run_ffn(te, xi, tv, xs, W1, b1r, W2)

    pos_flat = pos_tok.reshape(T * TOPK)
    y = _run_combine(ys, pos_flat, b2)
    return y


# speedup vs baseline: 1.0049x; 1.0049x over previous
import functools

import jax
import jax.numpy as jnp
from jax import lax
from jax.experimental import pallas as pl
from jax.experimental.pallas import tpu as pltpu
from jax.experimental.pallas import tpu_sc as plsc

T = 2048
IN_F = 2048
OUT_F = 8192
BLK = 512
NB = OUT_F // BLK
TOPK = NB // 4

RT = 256
NT = (T * TOPK + NB * RT) // RT
P_MAX = NT * RT
NTP = 64

NW = 32
TPW = T // NW
CCH = 32
TPC = 4

_HI = jax.lax.Precision.HIGHEST


def _meta_body(prob_ref, pos_tok_ref, te_ref, xi_ref, tv_ref):
    prob = prob_ref[...]
    ids_e = lax.broadcasted_iota(jnp.int32, (T, NB), 1)

    cols = []
    for e in range(NB):
        pn = prob[:, e:e + 1]
        beats = (prob > pn) | ((prob == pn) & (ids_e < e))
        cnt = jnp.sum(beats.astype(jnp.float32), axis=1, keepdims=True)
        cols.append((cnt < TOPK).astype(jnp.float32))
    maskf = jnp.concatenate(cols, axis=1)
    maskb = maskf > 0.5

    TC_ = 256
    NCHK = T // TC_
    r_i = lax.broadcasted_iota(jnp.int32, (TC_, TC_), 0)
    c_i = lax.broadcasted_iota(jnp.int32, (TC_, TC_), 1)
    tril_c = (c_i < r_i).astype(jnp.float32)
    ones_row_c = jnp.ones((1, TC_), jnp.float32)
    rank_chunks = []
    running = jnp.zeros((1, NB), jnp.float32)
    for c in range(NCHK):
        mc = maskf[c * TC_:(c + 1) * TC_, :]
        local = lax.dot_general(tril_c, mc, (((1,), (0,)), ((), ())),
                                precision=_HI)
        rank_chunks.append(running + local)
        running = running + lax.dot_general(ones_row_c, mc,
                                            (((1,), (0,)), ((), ())),
                                            precision=_HI)
    rank = jnp.concatenate(rank_chunks, axis=0)
    counts = running
    pc = jnp.floor((counts + (RT - 1)) / RT) * RT

    re = lax.broadcasted_iota(jnp.int32, (NB, NB), 0)
    ce = lax.broadcasted_iota(jnp.int32, (NB, NB), 1)
    l16s = (re < ce).astype(jnp.float32)
    starts = lax.dot_general(pc, l16s, (((1,), (0,)), ((), ())),
                             precision=_HI)
    ends = starts + pc

    p_te = starts + rank

    l16i = (re <= ce).astype(jnp.float32)
    ordm = lax.dot_general(maskf, l16i, (((1,), (0,)), ((), ())),
                           precision=_HI)

    pcols = []
    for j in range(TOPK):
        selj = maskb & (ordm == (j + 1))
        pcols.append(jnp.sum(jnp.where(selj, p_te, 0.0), axis=1, keepdims=True))
    pos_tok = jnp.concatenate(pcols, axis=1)
    pos_tok_ref[...] = pos_tok.astype(jnp.int32)

    u = jnp.sum(pc, axis=1, keepdims=True) / RT
    it = lax.broadcasted_iota(jnp.int32, (NTP, NB), 0).astype(jnp.float32)
    texp_raw = jnp.sum((it * RT >= ends).astype(jnp.float32),
                       axis=1, keepdims=True)
    texp_last = jnp.sum(((u - 1.0) * RT >= ends).astype(jnp.float32),
                        axis=1, keepdims=True)
    itcol = lax.broadcasted_iota(jnp.int32, (NTP, 1), 0).astype(jnp.float32)
    valid = itcol < u
    te_ref[...] = jnp.where(valid, texp_raw, texp_last).astype(jnp.int32)
    xi_ref[...] = jnp.minimum(itcol, u - 1.0).astype(jnp.int32)
    tv_ref[...] = valid.astype(jnp.int32)


def _run_meta(prob):
    return pl.pallas_call(
        _meta_body,
        out_shape=[
            jax.ShapeDtypeStruct((T, TOPK), jnp.int32),
            jax.ShapeDtypeStruct((NTP, 1), jnp.int32),
            jax.ShapeDtypeStruct((NTP, 1), jnp.int32),
            jax.ShapeDtypeStruct((NTP, 1), jnp.int32),
        ],
    )(prob)


def _scatter_body(x_hbm, pos_t_hbm, xs_hbm, xb0, xb1, ib0, ib1, lsem, ssem):
    wid = lax.axis_index("s") * 2 + lax.axis_index("c")
    base = wid * TPW
    pltpu.async_copy(x_hbm.at[pl.ds(base, CCH)], xb0, lsem)
    pltpu.async_copy(x_hbm.at[pl.ds(base + CCH, CCH)], xb1, lsem)
    for j in range(TOPK):
        pltpu.sync_copy(pos_t_hbm.at[j, pl.ds(base, CCH)], ib0.at[j])
        pltpu.sync_copy(pos_t_hbm.at[j, pl.ds(base + CCH, CCH)], ib1.at[j])
    pltpu.make_async_copy(x_hbm.at[pl.ds(base, CCH)], xb0, lsem).wait()
    pltpu.make_async_copy(x_hbm.at[pl.ds(base + CCH, CCH)], xb1, lsem).wait()
    for j in range(TOPK):
        pltpu.async_copy(xb0, xs_hbm.at[ib0.at[j]], ssem)
        pltpu.async_copy(xb1, xs_hbm.at[ib1.at[j]], ssem)
    for j in range(TOPK):
        pltpu.make_async_copy(xb0, xs_hbm.at[ib0.at[j]], ssem).wait()
        pltpu.make_async_copy(xb1, xs_hbm.at[ib1.at[j]], ssem).wait()


def _run_scatter(xp, pos_t):
    mesh = plsc.VectorSubcoreMesh(core_axis_name="c", subcore_axis_name="s")
    f = functools.partial(
        pl.kernel,
        out_type=jax.ShapeDtypeStruct((P_MAX, IN_F // 2), jnp.float32),
        mesh=mesh,
        scratch_types=[
            pltpu.VMEM((CCH, IN_F // 2), jnp.float32),
            pltpu.VMEM((CCH, IN_F // 2), jnp.float32),
            pltpu.VMEM((TOPK, CCH), jnp.int32),
            pltpu.VMEM((TOPK, CCH), jnp.int32),
            pltpu.SemaphoreType.DMA,
            pltpu.SemaphoreType.DMA,
        ],
    )(_scatter_body)
    return f(xp, pos_t)


def _ffn_body(te_ref, xi_ref, tv_ref, xs_ref, w1_ref, b1_ref, w2_ref, ys_ref):
    i = pl.program_id(0)

    @pl.when(tv_ref[i] == 1)
    def _():
        xt = xs_ref[...].astype(jnp.float32)
        h = lax.dot_general(xt, w1_ref[...], (((1,), (1,)), ((), ())),
                            preferred_element_type=jnp.float32)
        h = h + b1_ref[0]
        g = jax.nn.gelu(h)
        ys_ref[...] = lax.dot_general(g, w2_ref[...], (((1,), (1,)), ((), ())),
                                      preferred_element_type=jnp.float32)


def _run_ffn(te, xi, tv, xs, W1, b1r, W2):
    grid_spec = pltpu.PrefetchScalarGridSpec(
        num_scalar_prefetch=3,
        grid=(NT,),
        in_specs=[
            pl.BlockSpec((RT, IN_F), lambda i, te, xi, tv: (xi[i], 0)),
            pl.BlockSpec((BLK, IN_F), lambda i, te, xi, tv: (te[i], 0)),
            pl.BlockSpec((1, 1, BLK), lambda i, te, xi, tv: (te[i], 0, 0)),
            pl.BlockSpec((IN_F, BLK), lambda i, te, xi, tv: (0, te[i])),
        ],
        out_specs=pl.BlockSpec((RT, IN_F), lambda i, te, xi, tv: (xi[i], 0)),
    )
    return pl.pallas_call(
        _ffn_body,
        grid_spec=grid_spec,
        out_shape=jax.ShapeDtypeStruct((P_MAX, IN_F), jnp.float32),
        compiler_params=pltpu.CompilerParams(
            dimension_semantics=("arbitrary",),
        ),
    )(te, xi, tv, xs, W1, b1r, W2)


def _accum(rows, acc, b2v):
    def col(ci, carry):
        off = ci * 16
        for i in range(TPC):
            v = b2v[pl.ds(off, 16)]
            for j in range(TOPK):
                v = v + rows[TOPK * i + j, pl.ds(off, 16)]
            acc[i, pl.ds(off, 16)] = v
        return carry

    lax.fori_loop(0, IN_F // 16, col, 0, unroll=4)


def _combine_body(ys_hbm, pos_flat_hbm, b2_hbm, y_hbm, rows0, rows1, acc,
                  idx0, idx1, b2v, sem0, sem1):
    wid = lax.axis_index("s") * 2 + lax.axis_index("c")
    base = wid * TPW
    nch = TPW // TPC
    pltpu.sync_copy(b2_hbm, b2v)
    pltpu.sync_copy(pos_flat_hbm.at[pl.ds(base * TOPK, TPC * TOPK)], idx0)
    pltpu.async_copy(ys_hbm.at[idx0], rows0, sem0)

    def pair(m, carry):
        ta = base + (2 * m) * TPC
        tb = base + (2 * m + 1) * TPC
        pltpu.sync_copy(pos_flat_hbm.at[pl.ds(tb * TOPK, TPC * TOPK)], idx1)
        pltpu.async_copy(ys_hbm.at[idx1], rows1, sem1)
        pltpu.make_async_copy(ys_hbm.at[idx0], rows0, sem0).wait()
        _accum(rows0, acc, b2v)
        pltpu.sync_copy(acc, y_hbm.at[pl.ds(ta, TPC)])
        @pl.when(m + 1 < nch // 2)
        def _():
            tn = base + (2 * m + 2) * TPC
            pltpu.sync_copy(pos_flat_hbm.at[pl.ds(tn * TOPK, TPC * TOPK)],
                            idx0)
            pltpu.async_copy(ys_hbm.at[idx0], rows0, sem0)
        pltpu.make_async_copy(ys_hbm.at[idx1], rows1, sem1).wait()
        _accum(rows1, acc, b2v)
        pltpu.sync_copy(acc, y_hbm.at[pl.ds(tb, TPC)])
        return carry

    lax.fori_loop(0, nch // 2, pair, 0)


def _run_combine(ys, pos_flat, b2):
    mesh = plsc.VectorSubcoreMesh(core_axis_name="c", subcore_axis_name="s")
    f = functools.partial(
        pl.kernel,
        out_type=jax.ShapeDtypeStruct((T, IN_F), jnp.float32),
        mesh=mesh,
        scratch_types=[
            pltpu.VMEM((TPC * TOPK, IN_F), jnp.float32),
            pltpu.VMEM((TPC * TOPK, IN_F), jnp.float32),
            pltpu.VMEM((TPC, IN_F), jnp.float32),
            pltpu.VMEM((TPC * TOPK,), jnp.int32),
            pltpu.VMEM((TPC * TOPK,), jnp.int32),
            pltpu.VMEM((IN_F,), jnp.float32),
            pltpu.SemaphoreType.DMA,
            pltpu.SemaphoreType.DMA,
        ],
    )(_combine_body)
    return f(ys, pos_flat, b2)


def kernel(x, Wr, br, W1, b1, W2, b2):
    logits = x @ Wr.T + br[None, :]
    prob = jax.nn.softmax(logits, axis=-1)

    pos_tok, te, xi, tv = _run_meta(prob)
    pos_t = pos_tok.T
    te = te.reshape(NTP)
    xi = xi.reshape(NTP)
    tv = tv.reshape(NTP)

    xb = x.astype(jnp.bfloat16)
    xp = lax.bitcast_convert_type(xb.reshape(T, IN_F // 2, 2), jnp.float32)
    xsp = _run_scatter(xp, pos_t)
    xs = lax.bitcast_convert_type(xsp, jnp.bfloat16).reshape(P_MAX, IN_F)

    b1r = b1.reshape(NB, 1, BLK)
    ys = _run_ffn(te, xi, tv, xs, W1, b1r, W2)

    pos_flat = pos_tok.reshape(T * TOPK)
    y = _run_combine(ys, pos_flat, b2)
    return y


# speedup vs baseline: 2.6327x; 2.6199x over previous
import functools

import jax
import jax.numpy as jnp
from jax import lax
from jax.experimental import pallas as pl
from jax.experimental.pallas import tpu as pltpu
from jax.experimental.pallas import tpu_sc as plsc

T = 2048
IN_F = 2048
OUT_F = 8192
BLK = 512
NB = OUT_F // BLK
TOPK = NB // 4

RT = 256
NT = (T * TOPK + NB * RT) // RT
P_MAX = NT * RT
NTP = 64

NW = 32
TPW = T // NW
CCH = 16
TPC = 4

_HI = jax.lax.Precision.HIGHEST


def _meta_body(prob_ref, pos_tok_ref, te_ref, xi_ref, tv_ref):
    prob = prob_ref[...]
    ids_e = lax.broadcasted_iota(jnp.int32, (T, NB), 1)

    cols = []
    for e in range(NB):
        pn = prob[:, e:e + 1]
        beats = (prob > pn) | ((prob == pn) & (ids_e < e))
        cnt = jnp.sum(beats.astype(jnp.float32), axis=1, keepdims=True)
        cols.append((cnt < TOPK).astype(jnp.float32))
    maskf = jnp.concatenate(cols, axis=1)
    maskb = maskf > 0.5

    TC_ = 256
    NCHK = T // TC_
    r_i = lax.broadcasted_iota(jnp.int32, (TC_, TC_), 0)
    c_i = lax.broadcasted_iota(jnp.int32, (TC_, TC_), 1)
    tril_c = (c_i < r_i).astype(jnp.float32)
    ones_row_c = jnp.ones((1, TC_), jnp.float32)
    rank_chunks = []
    running = jnp.zeros((1, NB), jnp.float32)
    for c in range(NCHK):
        mc = maskf[c * TC_:(c + 1) * TC_, :]
        local = lax.dot_general(tril_c, mc, (((1,), (0,)), ((), ())),
                                precision=_HI)
        rank_chunks.append(running + local)
        running = running + lax.dot_general(ones_row_c, mc,
                                            (((1,), (0,)), ((), ())),
                                            precision=_HI)
    rank = jnp.concatenate(rank_chunks, axis=0)
    counts = running
    pc = jnp.floor((counts + (RT - 1)) / RT) * RT

    re = lax.broadcasted_iota(jnp.int32, (NB, NB), 0)
    ce = lax.broadcasted_iota(jnp.int32, (NB, NB), 1)
    l16s = (re < ce).astype(jnp.float32)
    starts = lax.dot_general(pc, l16s, (((1,), (0,)), ((), ())),
                             precision=_HI)
    ends = starts + pc

    p_te = starts + rank

    l16i = (re <= ce).astype(jnp.float32)
    ordm = lax.dot_general(maskf, l16i, (((1,), (0,)), ((), ())),
                           precision=_HI)

    pcols = []
    for j in range(TOPK):
        selj = maskb & (ordm == (j + 1))
        pcols.append(jnp.sum(jnp.where(selj, p_te, 0.0), axis=1, keepdims=True))
    pos_tok = jnp.concatenate(pcols, axis=1)
    pos_tok_ref[...] = pos_tok.astype(jnp.int32)

    u = jnp.sum(pc, axis=1, keepdims=True) / RT
    it = lax.broadcasted_iota(jnp.int32, (NTP, NB), 0).astype(jnp.float32)
    texp_raw = jnp.sum((it * RT >= ends).astype(jnp.float32),
                       axis=1, keepdims=True)
    texp_last = jnp.sum(((u - 1.0) * RT >= ends).astype(jnp.float32),
                        axis=1, keepdims=True)
    itcol = lax.broadcasted_iota(jnp.int32, (NTP, 1), 0).astype(jnp.float32)
    valid = itcol < u
    te_ref[...] = jnp.where(valid, texp_raw, texp_last).astype(jnp.int32)
    xi_ref[...] = jnp.minimum(itcol, u - 1.0).astype(jnp.int32)
    tv_ref[...] = valid.astype(jnp.int32)


def _run_meta(prob):
    return pl.pallas_call(
        _meta_body,
        out_shape=[
            jax.ShapeDtypeStruct((T, TOPK), jnp.int32),
            jax.ShapeDtypeStruct((NTP, 1), jnp.int32),
            jax.ShapeDtypeStruct((NTP, 1), jnp.int32),
            jax.ShapeDtypeStruct((NTP, 1), jnp.int32),
        ],
    )(prob)


def _scatter_body(x_hbm, pos_t_hbm, xs_hbm, xb0, xb1, ib0, ib1, lsem, s0, s1):
    wid = lax.axis_index("s") * 2 + lax.axis_index("c")
    base = wid * TPW
    nch = TPW // CCH
    bufs, ibs, sems = (xb0, xb1), (ib0, ib1), (s0, s1)
    for c in range(nch):
        b, ib, sem = bufs[c % 2], ibs[c % 2], sems[c % 2]
        if c >= 2:
            for j in range(TOPK):
                pltpu.make_async_copy(b, xs_hbm.at[ib.at[j]], sem).wait()
        pltpu.async_copy(x_hbm.at[pl.ds(base + c * CCH, CCH)], b, lsem)
        for j in range(TOPK):
            pltpu.sync_copy(pos_t_hbm.at[j, pl.ds(base + c * CCH, CCH)],
                            ib.at[j])
        pltpu.make_async_copy(x_hbm.at[pl.ds(base + c * CCH, CCH)], b,
                              lsem).wait()
        for j in range(TOPK):
            pltpu.async_copy(b, xs_hbm.at[ib.at[j]], sem)
    for c in (nch - 2, nch - 1):
        b, ib, sem = bufs[c % 2], ibs[c % 2], sems[c % 2]
        for j in range(TOPK):
            pltpu.make_async_copy(b, xs_hbm.at[ib.at[j]], sem).wait()


def _run_scatter(x, pos_t):
    mesh = plsc.VectorSubcoreMesh(core_axis_name="c", subcore_axis_name="s")
    f = functools.partial(
        pl.kernel,
        out_type=jax.ShapeDtypeStruct((P_MAX, IN_F), jnp.float32),
        mesh=mesh,
        scratch_types=[
            pltpu.VMEM((CCH, IN_F), jnp.float32),
            pltpu.VMEM((CCH, IN_F), jnp.float32),
            pltpu.VMEM((TOPK, CCH), jnp.int32),
            pltpu.VMEM((TOPK, CCH), jnp.int32),
            pltpu.SemaphoreType.DMA,
            pltpu.SemaphoreType.DMA,
            pltpu.SemaphoreType.DMA,
        ],
    )(_scatter_body)
    return f(x, pos_t)


def _ffn_body(te_ref, xi_ref, tv_ref, xs_ref, w1_ref, b1_ref, w2_ref, ys_ref):
    i = pl.program_id(0)

    @pl.when(tv_ref[i] == 1)
    def _():
        xt = xs_ref[...].astype(jnp.bfloat16)
        h = lax.dot_general(xt, w1_ref[...], (((1,), (1,)), ((), ())),
                            preferred_element_type=jnp.float32)
        h = h + b1_ref[0]
        g = jax.nn.gelu(h).astype(jnp.bfloat16)
        ys_ref[...] = lax.dot_general(g, w2_ref[...], (((1,), (1,)), ((), ())),
                                      preferred_element_type=jnp.float32)


def _run_ffn(te, xi, tv, xs, W1, b1r, W2):
    grid_spec = pltpu.PrefetchScalarGridSpec(
        num_scalar_prefetch=3,
        grid=(NT,),
        in_specs=[
            pl.BlockSpec((RT, IN_F), lambda i, te, xi, tv: (xi[i], 0)),
            pl.BlockSpec((BLK, IN_F), lambda i, te, xi, tv: (te[i], 0)),
            pl.BlockSpec((1, 1, BLK), lambda i, te, xi, tv: (te[i], 0, 0)),
            pl.BlockSpec((IN_F, BLK), lambda i, te, xi, tv: (0, te[i])),
        ],
        out_specs=pl.BlockSpec((RT, IN_F), lambda i, te, xi, tv: (xi[i], 0)),
    )
    return pl.pallas_call(
        _ffn_body,
        grid_spec=grid_spec,
        out_shape=jax.ShapeDtypeStruct((P_MAX, IN_F), jnp.float32),
        compiler_params=pltpu.CompilerParams(
            dimension_semantics=("arbitrary",),
        ),
    )(te, xi, tv, xs, W1, b1r, W2)


def _accum(rows, acc, b2v):
    def col(ci, carry):
        off = ci * 16
        for i in range(TPC):
            v = b2v[pl.ds(off, 16)]
            for j in range(TOPK):
                v = v + rows[TOPK * i + j, pl.ds(off, 16)]
            acc[i, pl.ds(off, 16)] = v
        return carry

    lax.fori_loop(0, IN_F // 16, col, 0, unroll=4)


def _combine_body(ys_hbm, pos_flat_hbm, b2_hbm, y_hbm, rows0, rows1, acc,
                  idx0, idx1, b2v, sem0, sem1):
    wid = lax.axis_index("s") * 2 + lax.axis_index("c")
    base = wid * TPW
    nch = TPW // TPC
    pltpu.sync_copy(b2_hbm, b2v)
    pltpu.sync_copy(pos_flat_hbm.at[pl.ds(base * TOPK, TPC * TOPK)], idx0)
    pltpu.async_copy(ys_hbm.at[idx0], rows0, sem0)

    def pair(m, carry):
        ta = base + (2 * m) * TPC
        tb = base + (2 * m + 1) * TPC
        pltpu.sync_copy(pos_flat_hbm.at[pl.ds(tb * TOPK, TPC * TOPK)], idx1)
        pltpu.async_copy(ys_hbm.at[idx1], rows1, sem1)
        pltpu.make_async_copy(ys_hbm.at[idx0], rows0, sem0).wait()
        _accum(rows0, acc, b2v)
        pltpu.sync_copy(acc, y_hbm.at[pl.ds(ta, TPC)])
        @pl.when(m + 1 < nch // 2)
        def _():
            tn = base + (2 * m + 2) * TPC
            pltpu.sync_copy(pos_flat_hbm.at[pl.ds(tn * TOPK, TPC * TOPK)],
                            idx0)
            pltpu.async_copy(ys_hbm.at[idx0], rows0, sem0)
        pltpu.make_async_copy(ys_hbm.at[idx1], rows1, sem1).wait()
        _accum(rows1, acc, b2v)
        pltpu.sync_copy(acc, y_hbm.at[pl.ds(tb, TPC)])
        return carry

    lax.fori_loop(0, nch // 2, pair, 0)


def _run_combine(ys, pos_flat, b2):
    mesh = plsc.VectorSubcoreMesh(core_axis_name="c", subcore_axis_name="s")
    f = functools.partial(
        pl.kernel,
        out_type=jax.ShapeDtypeStruct((T, IN_F), jnp.float32),
        mesh=mesh,
        scratch_types=[
            pltpu.VMEM((TPC * TOPK, IN_F), jnp.float32),
            pltpu.VMEM((TPC * TOPK, IN_F), jnp.float32),
            pltpu.VMEM((TPC, IN_F), jnp.float32),
            pltpu.VMEM((TPC * TOPK,), jnp.int32),
            pltpu.VMEM((TPC * TOPK,), jnp.int32),
            pltpu.VMEM((IN_F,), jnp.float32),
            pltpu.SemaphoreType.DMA,
            pltpu.SemaphoreType.DMA,
        ],
    )(_combine_body)
    return f(ys, pos_flat, b2)


def kernel(x, Wr, br, W1, b1, W2, b2):
    logits = x @ Wr.T + br[None, :]
    prob = jax.nn.softmax(logits, axis=-1)

    pos_tok, te, xi, tv = _run_meta(prob)
    pos_t = pos_tok.T
    te = te.reshape(NTP)
    xi = xi.reshape(NTP)
    tv = tv.reshape(NTP)

    xs = _run_scatter(x, pos_t)

    b1r = b1.reshape(NB, 1, BLK)
    ys = _run_ffn(te, xi, tv, xs, W1.astype(jnp.bfloat16), b1r,
                  W2.astype(jnp.bfloat16))

    pos_flat = pos_tok.reshape(T * TOPK)
    y = _run_combine(ys, pos_flat, b2)
    return y


# speedup vs baseline: 3.0271x; 1.1498x over previous
import functools

import jax
import jax.numpy as jnp
from jax import lax
from jax.experimental import pallas as pl
from jax.experimental.pallas import tpu as pltpu
from jax.experimental.pallas import tpu_sc as plsc

T = 2048
IN_F = 2048
OUT_F = 8192
BLK = 512
NB = OUT_F // BLK
TOPK = NB // 4

RT = 256
NT = (T * TOPK + NB * RT) // RT
P_MAX = NT * RT
NTP = 64

NW = 32
TPW = T // NW
CCH = 16
TPC = 4

_HI = jax.lax.Precision.HIGHEST


def _meta_body(prob_ref, pos_tok_ref, te_ref, xi_ref, tv_ref):
    prob = prob_ref[...]
    ids_e = lax.broadcasted_iota(jnp.int32, (T, NB), 1)

    cols = []
    for e in range(NB):
        pn = prob[:, e:e + 1]
        beats = (prob > pn) | ((prob == pn) & (ids_e < e))
        cnt = jnp.sum(beats.astype(jnp.float32), axis=1, keepdims=True)
        cols.append((cnt < TOPK).astype(jnp.float32))
    maskf = jnp.concatenate(cols, axis=1)
    maskb = maskf > 0.5

    TC_ = 256
    NCHK = T // TC_
    r_i = lax.broadcasted_iota(jnp.int32, (TC_, TC_), 0)
    c_i = lax.broadcasted_iota(jnp.int32, (TC_, TC_), 1)
    tril_c = (c_i < r_i).astype(jnp.float32)
    ones_row_c = jnp.ones((1, TC_), jnp.float32)
    rank_chunks = []
    running = jnp.zeros((1, NB), jnp.float32)
    for c in range(NCHK):
        mc = maskf[c * TC_:(c + 1) * TC_, :]
        local = lax.dot_general(tril_c, mc, (((1,), (0,)), ((), ())),
                                precision=_HI)
        rank_chunks.append(running + local)
        running = running + lax.dot_general(ones_row_c, mc,
                                            (((1,), (0,)), ((), ())),
                                            precision=_HI)
    rank = jnp.concatenate(rank_chunks, axis=0)
    counts = running
    pc = jnp.floor((counts + (RT - 1)) / RT) * RT

    re = lax.broadcasted_iota(jnp.int32, (NB, NB), 0)
    ce = lax.broadcasted_iota(jnp.int32, (NB, NB), 1)
    l16s = (re < ce).astype(jnp.float32)
    starts = lax.dot_general(pc, l16s, (((1,), (0,)), ((), ())),
                             precision=_HI)
    ends = starts + pc

    p_te = starts + rank

    l16i = (re <= ce).astype(jnp.float32)
    ordm = lax.dot_general(maskf, l16i, (((1,), (0,)), ((), ())),
                           precision=_HI)

    pcols = []
    for j in range(TOPK):
        selj = maskb & (ordm == (j + 1))
        pcols.append(jnp.sum(jnp.where(selj, p_te, 0.0), axis=1, keepdims=True))
    pos_tok = jnp.concatenate(pcols, axis=1)
    pos_tok_ref[...] = pos_tok.astype(jnp.int32)

    u = jnp.sum(pc, axis=1, keepdims=True) / RT
    it = lax.broadcasted_iota(jnp.int32, (NTP, NB), 0).astype(jnp.float32)
    texp_raw = jnp.sum((it * RT >= ends).astype(jnp.float32),
                       axis=1, keepdims=True)
    texp_last = jnp.sum(((u - 1.0) * RT >= ends).astype(jnp.float32),
                        axis=1, keepdims=True)
    itcol = lax.broadcasted_iota(jnp.int32, (NTP, 1), 0).astype(jnp.float32)
    valid = itcol < u
    te_ref[...] = jnp.where(valid, texp_raw, texp_last).astype(jnp.int32)
    xi_ref[...] = jnp.minimum(itcol, u - 1.0).astype(jnp.int32)
    tv_ref[...] = valid.astype(jnp.int32)


def _run_meta(prob):
    return pl.pallas_call(
        _meta_body,
        out_shape=[
            jax.ShapeDtypeStruct((T, TOPK), jnp.int32),
            jax.ShapeDtypeStruct((NTP, 1), jnp.int32),
            jax.ShapeDtypeStruct((NTP, 1), jnp.int32),
            jax.ShapeDtypeStruct((NTP, 1), jnp.int32),
        ],
    )(prob)


def _scatter_body(x_hbm, pos_t_hbm, xs_hbm, xb0, xb1, ib0, ib1, lsem, s0, s1):
    wid = lax.axis_index("s") * 2 + lax.axis_index("c")
    base = wid * TPW
    nch = TPW // CCH
    bufs, ibs, sems = (xb0, xb1), (ib0, ib1), (s0, s1)
    for c in range(nch):
        b, ib, sem = bufs[c % 2], ibs[c % 2], sems[c % 2]
        if c >= 2:
            for j in range(TOPK):
                pltpu.make_async_copy(b, xs_hbm.at[ib.at[j]], sem).wait()
        pltpu.async_copy(x_hbm.at[pl.ds(base + c * CCH, CCH)], b, lsem)
        for j in range(TOPK):
            pltpu.sync_copy(pos_t_hbm.at[j, pl.ds(base + c * CCH, CCH)],
                            ib.at[j])
        pltpu.make_async_copy(x_hbm.at[pl.ds(base + c * CCH, CCH)], b,
                              lsem).wait()
        for j in range(TOPK):
            pltpu.async_copy(b, xs_hbm.at[ib.at[j]], sem)
    for c in (nch - 2, nch - 1):
        b, ib, sem = bufs[c % 2], ibs[c % 2], sems[c % 2]
        for j in range(TOPK):
            pltpu.make_async_copy(b, xs_hbm.at[ib.at[j]], sem).wait()


def _run_scatter(x, pos_t):
    mesh = plsc.VectorSubcoreMesh(core_axis_name="c", subcore_axis_name="s")
    f = functools.partial(
        pl.kernel,
        out_type=jax.ShapeDtypeStruct((P_MAX, IN_F), jnp.float32),
        mesh=mesh,
        scratch_types=[
            pltpu.VMEM((CCH, IN_F), jnp.float32),
            pltpu.VMEM((CCH, IN_F), jnp.float32),
            pltpu.VMEM((TOPK, CCH), jnp.int32),
            pltpu.VMEM((TOPK, CCH), jnp.int32),
            pltpu.SemaphoreType.DMA,
            pltpu.SemaphoreType.DMA,
            pltpu.SemaphoreType.DMA,
        ],
    )(_scatter_body)
    return f(x, pos_t)


def _ffn_body(te_ref, xi_ref, tv_ref, xs_ref, w1_ref, b1_ref, w2_ref, ys_ref):
    i = pl.program_id(0)

    @pl.when(tv_ref[i] == 1)
    def _():
        xt = xs_ref[...].astype(jnp.bfloat16)
        h = lax.dot_general(xt, w1_ref[...], (((1,), (1,)), ((), ())),
                            preferred_element_type=jnp.float32)
        h = h + b1_ref[0]
        g = jax.nn.gelu(h).astype(jnp.bfloat16)
        y = lax.dot_general(g, w2_ref[...], (((1,), (1,)), ((), ())),
                            preferred_element_type=jnp.float32)
        vl = y[:, :IN_F // 2].astype(jnp.bfloat16).astype(jnp.float32)
        vh = y[:, IN_F // 2:].astype(jnp.bfloat16).astype(jnp.float32)
        il = lax.shift_right_logical(
            lax.bitcast_convert_type(vl, jnp.int32), 16)
        ih = lax.bitcast_convert_type(vh, jnp.int32)
        ys_ref[...] = lax.bitcast_convert_type(il | ih, jnp.float32)


def _run_ffn(te, xi, tv, xs, W1, b1r, W2):
    grid_spec = pltpu.PrefetchScalarGridSpec(
        num_scalar_prefetch=3,
        grid=(NT,),
        in_specs=[
            pl.BlockSpec((RT, IN_F), lambda i, te, xi, tv: (xi[i], 0)),
            pl.BlockSpec((BLK, IN_F), lambda i, te, xi, tv: (te[i], 0)),
            pl.BlockSpec((1, 1, BLK), lambda i, te, xi, tv: (te[i], 0, 0)),
            pl.BlockSpec((IN_F, BLK), lambda i, te, xi, tv: (0, te[i])),
        ],
        out_specs=pl.BlockSpec((RT, IN_F // 2),
                               lambda i, te, xi, tv: (xi[i], 0)),
    )
    return pl.pallas_call(
        _ffn_body,
        grid_spec=grid_spec,
        out_shape=jax.ShapeDtypeStruct((P_MAX, IN_F // 2), jnp.float32),
        compiler_params=pltpu.CompilerParams(
            dimension_semantics=("arbitrary",),
        ),
    )(te, xi, tv, xs, W1, b1r, W2)


def _accum(rows, acc, b2v):
    def col(ci, carry):
        off = ci * 16
        for i in range(TPC):
            vl = b2v[pl.ds(off, 16)]
            vh = b2v[pl.ds(IN_F // 2 + off, 16)]
            for j in range(TOPK):
                p = lax.bitcast_convert_type(
                    rows[TOPK * i + j, pl.ds(off, 16)], jnp.int32)
                vl = vl + lax.bitcast_convert_type(
                    lax.shift_left(p, 16), jnp.float32)
                vh = vh + lax.bitcast_convert_type(
                    p & jnp.int32(-65536), jnp.float32)
            acc[i, pl.ds(off, 16)] = vl
            acc[i, pl.ds(IN_F // 2 + off, 16)] = vh
        return carry

    lax.fori_loop(0, IN_F // 2 // 16, col, 0, unroll=4)


def _combine_body(ys_hbm, pos_flat_hbm, b2_hbm, y_hbm, rows0, rows1, acc,
                  idx0, idx1, b2v, sem0, sem1):
    wid = lax.axis_index("s") * 2 + lax.axis_index("c")
    base = wid * TPW
    nch = TPW // TPC
    pltpu.sync_copy(b2_hbm, b2v)
    pltpu.sync_copy(pos_flat_hbm.at[pl.ds(base * TOPK, TPC * TOPK)], idx0)
    pltpu.async_copy(ys_hbm.at[idx0], rows0, sem0)

    def pair(m, carry):
        ta = base + (2 * m) * TPC
        tb = base + (2 * m + 1) * TPC
        pltpu.sync_copy(pos_flat_hbm.at[pl.ds(tb * TOPK, TPC * TOPK)], idx1)
        pltpu.async_copy(ys_hbm.at[idx1], rows1, sem1)
        pltpu.make_async_copy(ys_hbm.at[idx0], rows0, sem0).wait()
        _accum(rows0, acc, b2v)
        pltpu.sync_copy(acc, y_hbm.at[pl.ds(ta, TPC)])
        @pl.when(m + 1 < nch // 2)
        def _():
            tn = base + (2 * m + 2) * TPC
            pltpu.sync_copy(pos_flat_hbm.at[pl.ds(tn * TOPK, TPC * TOPK)],
                            idx0)
            pltpu.async_copy(ys_hbm.at[idx0], rows0, sem0)
        pltpu.make_async_copy(ys_hbm.at[idx1], rows1, sem1).wait()
        _accum(rows1, acc, b2v)
        pltpu.sync_copy(acc, y_hbm.at[pl.ds(tb, TPC)])
        return carry

    lax.fori_loop(0, nch // 2, pair, 0)


def _run_combine(ys, pos_flat, b2):
    mesh = plsc.VectorSubcoreMesh(core_axis_name="c", subcore_axis_name="s")
    f = functools.partial(
        pl.kernel,
        out_type=jax.ShapeDtypeStruct((T, IN_F), jnp.float32),
        mesh=mesh,
        scratch_types=[
            pltpu.VMEM((TPC * TOPK, IN_F // 2), jnp.float32),
            pltpu.VMEM((TPC * TOPK, IN_F // 2), jnp.float32),
            pltpu.VMEM((TPC, IN_F), jnp.float32),
            pltpu.VMEM((TPC * TOPK,), jnp.int32),
            pltpu.VMEM((TPC * TOPK,), jnp.int32),
            pltpu.VMEM((IN_F,), jnp.float32),
            pltpu.SemaphoreType.DMA,
            pltpu.SemaphoreType.DMA,
        ],
    )(_combine_body)
    return f(ys, pos_flat, b2)


def kernel(x, Wr, br, W1, b1, W2, b2):
    logits = x @ Wr.T + br[None, :]
    prob = jax.nn.softmax(logits, axis=-1)

    pos_tok, te, xi, tv = _run_meta(prob)
    pos_t = pos_tok.T
    te = te.reshape(NTP)
    xi = xi.reshape(NTP)
    tv = tv.reshape(NTP)

    xs = _run_scatter(x, pos_t)

    b1r = b1.reshape(NB, 1, BLK)
    ys = _run_ffn(te, xi, tv, xs, W1.astype(jnp.bfloat16), b1r,
                  W2.astype(jnp.bfloat16))

    pos_flat = pos_tok.reshape(T * TOPK)
    y = _run_combine(ys, pos_flat, b2)
    return y


# speedup vs baseline: 3.6500x; 1.2058x over previous
import functools

import jax
import jax.numpy as jnp
from jax import lax
from jax.experimental import pallas as pl
from jax.experimental.pallas import tpu as pltpu
from jax.experimental.pallas import tpu_sc as plsc

T = 2048
IN_F = 2048
OUT_F = 8192
BLK = 512
NB = OUT_F // BLK
TOPK = NB // 4

RT = 256
NT = (T * TOPK + NB * RT) // RT
P_MAX = NT * RT
NTP = 64

NW = 32
TPW = T // NW
CCH = 32
TPC = 8

_HI = jax.lax.Precision.HIGHEST


def _meta_body(prob_ref, x_ref, pos_tok_ref, xp_ref, te_ref, xi_ref, tv_ref):
    xv = x_ref[...]
    vl = xv[:, :IN_F // 2].astype(jnp.bfloat16).astype(jnp.float32)
    vh = xv[:, IN_F // 2:].astype(jnp.bfloat16).astype(jnp.float32)
    il = lax.shift_right_logical(lax.bitcast_convert_type(vl, jnp.int32), 16)
    ih = lax.bitcast_convert_type(vh, jnp.int32)
    xp_ref[...] = lax.bitcast_convert_type(il | ih, jnp.float32)

    prob = prob_ref[...]
    ids_e = lax.broadcasted_iota(jnp.int32, (T, NB), 1)

    cols = []
    for e in range(NB):
        pn = prob[:, e:e + 1]
        beats = (prob > pn) | ((prob == pn) & (ids_e < e))
        cnt = jnp.sum(beats.astype(jnp.float32), axis=1, keepdims=True)
        cols.append((cnt < TOPK).astype(jnp.float32))
    maskf = jnp.concatenate(cols, axis=1)
    maskb = maskf > 0.5

    TC_ = 256
    NCHK = T // TC_
    r_i = lax.broadcasted_iota(jnp.int32, (TC_, TC_), 0)
    c_i = lax.broadcasted_iota(jnp.int32, (TC_, TC_), 1)
    tril_c = (c_i < r_i).astype(jnp.float32)
    ones_row_c = jnp.ones((1, TC_), jnp.float32)
    rank_chunks = []
    running = jnp.zeros((1, NB), jnp.float32)
    for c in range(NCHK):
        mc = maskf[c * TC_:(c + 1) * TC_, :]
        local = lax.dot_general(tril_c, mc, (((1,), (0,)), ((), ())),
                                precision=_HI)
        rank_chunks.append(running + local)
        running = running + lax.dot_general(ones_row_c, mc,
                                            (((1,), (0,)), ((), ())),
                                            precision=_HI)
    rank = jnp.concatenate(rank_chunks, axis=0)
    counts = running
    pc = jnp.floor((counts + (RT - 1)) / RT) * RT

    re = lax.broadcasted_iota(jnp.int32, (NB, NB), 0)
    ce = lax.broadcasted_iota(jnp.int32, (NB, NB), 1)
    l16s = (re < ce).astype(jnp.float32)
    starts = lax.dot_general(pc, l16s, (((1,), (0,)), ((), ())),
                             precision=_HI)
    ends = starts + pc

    p_te = starts + rank

    l16i = (re <= ce).astype(jnp.float32)
    ordm = lax.dot_general(maskf, l16i, (((1,), (0,)), ((), ())),
                           precision=_HI)

    pcols = []
    for j in range(TOPK):
        selj = maskb & (ordm == (j + 1))
        pcols.append(jnp.sum(jnp.where(selj, p_te, 0.0), axis=1, keepdims=True))
    pos_tok = jnp.concatenate(pcols, axis=1)
    pos_tok_ref[...] = pos_tok.astype(jnp.int32)

    u = jnp.sum(pc, axis=1, keepdims=True) / RT
    it = lax.broadcasted_iota(jnp.int32, (NTP, NB), 0).astype(jnp.float32)
    texp_raw = jnp.sum((it * RT >= ends).astype(jnp.float32),
                       axis=1, keepdims=True)
    texp_last = jnp.sum(((u - 1.0) * RT >= ends).astype(jnp.float32),
                        axis=1, keepdims=True)
    itcol = lax.broadcasted_iota(jnp.int32, (NTP, 1), 0).astype(jnp.float32)
    valid = itcol < u
    te_ref[...] = jnp.where(valid, texp_raw, texp_last).astype(jnp.int32)
    xi_ref[...] = jnp.minimum(itcol, u - 1.0).astype(jnp.int32)
    tv_ref[...] = valid.astype(jnp.int32)


def _run_meta(prob, x):
    return pl.pallas_call(
        _meta_body,
        out_shape=[
            jax.ShapeDtypeStruct((T, TOPK), jnp.int32),
            jax.ShapeDtypeStruct((T, IN_F // 2), jnp.float32),
            jax.ShapeDtypeStruct((NTP, 1), jnp.int32),
            jax.ShapeDtypeStruct((NTP, 1), jnp.int32),
            jax.ShapeDtypeStruct((NTP, 1), jnp.int32),
        ],
    )(prob, x)


def _scatter_body(x_hbm, pos_t_hbm, xs_hbm, xb0, xb1, ib0, ib1, lsem, s0, s1):
    wid = lax.axis_index("s") * 2 + lax.axis_index("c")
    base = wid * TPW
    nch = TPW // CCH
    bufs, ibs, sems = (xb0, xb1), (ib0, ib1), (s0, s1)
    for c in range(nch):
        b, ib, sem = bufs[c % 2], ibs[c % 2], sems[c % 2]
        if c >= 2:
            for j in range(TOPK):
                pltpu.make_async_copy(b, xs_hbm.at[ib.at[j]], sem).wait()
        pltpu.async_copy(x_hbm.at[pl.ds(base + c * CCH, CCH)], b, lsem)
        for j in range(TOPK):
            pltpu.sync_copy(pos_t_hbm.at[j, pl.ds(base + c * CCH, CCH)],
                            ib.at[j])
        pltpu.make_async_copy(x_hbm.at[pl.ds(base + c * CCH, CCH)], b,
                              lsem).wait()
        for j in range(TOPK):
            pltpu.async_copy(b, xs_hbm.at[ib.at[j]], sem)
    for c in (nch - 2, nch - 1):
        b, ib, sem = bufs[c % 2], ibs[c % 2], sems[c % 2]
        for j in range(TOPK):
            pltpu.make_async_copy(b, xs_hbm.at[ib.at[j]], sem).wait()


def _run_scatter(xp, pos_t):
    mesh = plsc.VectorSubcoreMesh(core_axis_name="c", subcore_axis_name="s")
    f = functools.partial(
        pl.kernel,
        out_type=jax.ShapeDtypeStruct((P_MAX, IN_F // 2), jnp.float32),
        mesh=mesh,
        scratch_types=[
            pltpu.VMEM((CCH, IN_F // 2), jnp.float32),
            pltpu.VMEM((CCH, IN_F // 2), jnp.float32),
            pltpu.VMEM((TOPK, CCH), jnp.int32),
            pltpu.VMEM((TOPK, CCH), jnp.int32),
            pltpu.SemaphoreType.DMA,
            pltpu.SemaphoreType.DMA,
            pltpu.SemaphoreType.DMA,
        ],
    )(_scatter_body)
    return f(xp, pos_t)


def _ffn_body(te_ref, xi_ref, tv_ref, xs_ref, w1_ref, b1_ref, w2_ref, ys_ref):
    i = pl.program_id(0)

    @pl.when(tv_ref[i] == 1)
    def _():
        p = lax.bitcast_convert_type(xs_ref[...], jnp.int32)
        xlo = lax.bitcast_convert_type(lax.shift_left(p, 16),
                                       jnp.float32).astype(jnp.bfloat16)
        xhi = lax.bitcast_convert_type(p & jnp.int32(-65536),
                                       jnp.float32).astype(jnp.bfloat16)
        w1 = w1_ref[...].astype(jnp.bfloat16)
        h = lax.dot_general(xlo, w1[:, :IN_F // 2],
                            (((1,), (1,)), ((), ())),
                            preferred_element_type=jnp.float32)
        h = h + lax.dot_general(xhi, w1[:, IN_F // 2:],
                                (((1,), (1,)), ((), ())),
                                preferred_element_type=jnp.float32)
        h = h + b1_ref[0]
        g = jax.nn.gelu(h).astype(jnp.bfloat16)
        y = lax.dot_general(g, w2_ref[...].astype(jnp.bfloat16),
                            (((1,), (1,)), ((), ())),
                            preferred_element_type=jnp.float32)
        vl = y[:, :IN_F // 2].astype(jnp.bfloat16).astype(jnp.float32)
        vh = y[:, IN_F // 2:].astype(jnp.bfloat16).astype(jnp.float32)
        il = lax.shift_right_logical(
            lax.bitcast_convert_type(vl, jnp.int32), 16)
        ih = lax.bitcast_convert_type(vh, jnp.int32)
        ys_ref[...] = lax.bitcast_convert_type(il | ih, jnp.float32)


def _run_ffn(te, xi, tv, xs, W1, b1r, W2):
    grid_spec = pltpu.PrefetchScalarGridSpec(
        num_scalar_prefetch=3,
        grid=(NT,),
        in_specs=[
            pl.BlockSpec((RT, IN_F // 2), lambda i, te, xi, tv: (xi[i], 0)),
            pl.BlockSpec((BLK, IN_F), lambda i, te, xi, tv: (te[i], 0)),
            pl.BlockSpec((1, 1, BLK), lambda i, te, xi, tv: (te[i], 0, 0)),
            pl.BlockSpec((IN_F, BLK), lambda i, te, xi, tv: (0, te[i])),
        ],
        out_specs=pl.BlockSpec((RT, IN_F // 2),
                               lambda i, te, xi, tv: (xi[i], 0)),
    )
    return pl.pallas_call(
        _ffn_body,
        grid_spec=grid_spec,
        out_shape=jax.ShapeDtypeStruct((P_MAX, IN_F // 2), jnp.float32),
        compiler_params=pltpu.CompilerParams(
            dimension_semantics=("arbitrary",),
        ),
    )(te, xi, tv, xs, W1, b1r, W2)


def _accum(rows, acc, b2v):
    def col(ci, carry):
        off = ci * 16
        for i in range(TPC):
            vl = b2v[pl.ds(off, 16)]
            vh = b2v[pl.ds(IN_F // 2 + off, 16)]
            for j in range(TOPK):
                p = lax.bitcast_convert_type(
                    rows[TOPK * i + j, pl.ds(off, 16)], jnp.int32)
                vl = vl + lax.bitcast_convert_type(
                    lax.shift_left(p, 16), jnp.float32)
                vh = vh + lax.bitcast_convert_type(
                    p & jnp.int32(-65536), jnp.float32)
            acc[i, pl.ds(off, 16)] = vl
            acc[i, pl.ds(IN_F // 2 + off, 16)] = vh
        return carry

    lax.fori_loop(0, IN_F // 2 // 16, col, 0, unroll=4)


def _combine_body(ys_hbm, pos_flat_hbm, b2_hbm, y_hbm, rows0, rows1, acc,
                  idx0, idx1, b2v, sem0, sem1):
    wid = lax.axis_index("s") * 2 + lax.axis_index("c")
    base = wid * TPW
    nch = TPW // TPC
    pltpu.sync_copy(b2_hbm, b2v)
    pltpu.sync_copy(pos_flat_hbm.at[pl.ds(base * TOPK, TPC * TOPK)], idx0)
    pltpu.async_copy(ys_hbm.at[idx0], rows0, sem0)

    def pair(m, carry):
        ta = base + (2 * m) * TPC
        tb = base + (2 * m + 1) * TPC
        pltpu.sync_copy(pos_flat_hbm.at[pl.ds(tb * TOPK, TPC * TOPK)], idx1)
        pltpu.async_copy(ys_hbm.at[idx1], rows1, sem1)
        pltpu.make_async_copy(ys_hbm.at[idx0], rows0, sem0).wait()
        _accum(rows0, acc, b2v)
        pltpu.sync_copy(acc, y_hbm.at[pl.ds(ta, TPC)])
        @pl.when(m + 1 < nch // 2)
        def _():
            tn = base + (2 * m + 2) * TPC
            pltpu.sync_copy(pos_flat_hbm.at[pl.ds(tn * TOPK, TPC * TOPK)],
                            idx0)
            pltpu.async_copy(ys_hbm.at[idx0], rows0, sem0)
        pltpu.make_async_copy(ys_hbm.at[idx1], rows1, sem1).wait()
        _accum(rows1, acc, b2v)
        pltpu.sync_copy(acc, y_hbm.at[pl.ds(tb, TPC)])
        return carry

    lax.fori_loop(0, nch // 2, pair, 0)


def _run_combine(ys, pos_flat, b2):
    mesh = plsc.VectorSubcoreMesh(core_axis_name="c", subcore_axis_name="s")
    f = functools.partial(
        pl.kernel,
        out_type=jax.ShapeDtypeStruct((T, IN_F), jnp.float32),
        mesh=mesh,
        scratch_types=[
            pltpu.VMEM((TPC * TOPK, IN_F // 2), jnp.float32),
            pltpu.VMEM((TPC * TOPK, IN_F // 2), jnp.float32),
            pltpu.VMEM((TPC, IN_F), jnp.float32),
            pltpu.VMEM((TPC * TOPK,), jnp.int32),
            pltpu.VMEM((TPC * TOPK,), jnp.int32),
            pltpu.VMEM((IN_F,), jnp.float32),
            pltpu.SemaphoreType.DMA,
            pltpu.SemaphoreType.DMA,
        ],
    )(_combine_body)
    return f(ys, pos_flat, b2)


def kernel(x, Wr, br, W1, b1, W2, b2):
    logits = x @ Wr.T + br[None, :]
    prob = jax.nn.softmax(logits, axis=-1)

    pos_tok, xp, te, xi, tv = _run_meta(prob, x)
    pos_t = pos_tok.T
    te = te.reshape(NTP)
    xi = xi.reshape(NTP)
    tv = tv.reshape(NTP)

    xs = _run_scatter(xp, pos_t)

    b1r = b1.reshape(NB, 1, BLK)
    ys = _run_ffn(te, xi, tv, xs, W1, b1r, W2)

    pos_flat = pos_tok.reshape(T * TOPK)
    y = _run_combine(ys, pos_flat, b2)
    return y
